# Initial kernel scaffold; baseline (speedup 1.0000x reference)
import jax
import jax.numpy as jnp
from jax.experimental import pallas as pl


def kernel(idx, embeddings):
    raise NotImplementedError("write your pallas kernel here")


# speedup vs baseline: 4.8285x; 4.8285x over previous
import functools

import jax
import jax.numpy as jnp
from jax import lax
from jax.experimental import pallas as pl
from jax.experimental.pallas import tpu as pltpu
from jax.experimental.pallas import tpu_sc as plsc

DIM = 32
SUB = 128
NSUB = 8
CHUNK = SUB * NSUB
LANES = 16


@functools.partial(jax.jit, static_argnums=(2, 3))
def _gather(idx2, table, n_rows, padding_idx):
    info = plsc.get_sparse_core_info()
    nw = info.num_cores * info.num_subcores
    n_w = n_rows // nw
    nch = n_w // CHUNK

    mesh = plsc.VectorSubcoreMesh(core_axis_name="c", subcore_axis_name="s")

    @functools.partial(
        pl.kernel,
        mesh=mesh,
        compiler_params=pltpu.CompilerParams(use_tc_tiling_on_sc=False),
        out_type=jax.ShapeDtypeStruct((n_rows, DIM), jnp.float32),
        scratch_types=[
            pltpu.VMEM((NSUB, SUB), jnp.int32),
            pltpu.VMEM((CHUNK, DIM), jnp.float32),
            pltpu.SemaphoreType.DMA,
        ],
    )
    def k(idx_hbm, table_hbm, out_hbm, idx_v, rows_v, sem):
        c = lax.axis_index("c")
        s = lax.axis_index("s")
        wid = s * info.num_cores + c
        base = wid * n_w

        def chunk_body(g, carry):
            row0 = pl.multiple_of(base + g * CHUNK, CHUNK)
            pltpu.sync_copy(
                idx_hbm.at[pl.ds(pl.multiple_of(row0 // SUB, NSUB), NSUB)],
                idx_v)
            copies = [
                pltpu.async_copy(
                    table_hbm.at[idx_v.at[j]],
                    rows_v.at[pl.ds(j * SUB, SUB)],
                    sem,
                )
                for j in range(NSUB)
            ]
            acc = idx_v[0, pl.ds(0, LANES)] == padding_idx
            for t in range(1, CHUNK // LANES):
                vec = idx_v[t // (SUB // LANES),
                            pl.ds((t % (SUB // LANES)) * LANES, LANES)]
                acc = acc | (vec == padding_idx)
            acc_i = jnp.where(acc, 1, 0)
            pad_cnt = acc_i[0]
            for lane in range(1, LANES):
                pad_cnt = pad_cnt | acc_i[lane]
            has_pad = pad_cnt > 0
            for cp in copies:
                cp.wait()

            @pl.when(has_pad)
            def _fix():
                zeros = jnp.zeros((LANES,), jnp.float32)

                def grp_fix(t, carry2):
                    j = t // (SUB // LANES)
                    off = (t % (SUB // LANES)) * LANES
                    vec = idx_v[j, pl.ds(off, LANES)]
                    for lane in range(LANES):
                        row = t * LANES + lane

                        @pl.when(vec[lane] == padding_idx)
                        def _z(row=row):
                            rows_v[row, pl.ds(0, LANES)] = zeros
                            rows_v[row, pl.ds(LANES, LANES)] = zeros

                    return carry2

                lax.fori_loop(0, CHUNK // LANES, grp_fix, 0)

            pltpu.sync_copy(rows_v, out_hbm.at[pl.ds(row0, CHUNK)])
            return carry

        lax.fori_loop(0, nch, chunk_body, 0)

    return k(idx2, table)


def kernel(idx, embeddings):
    b, t = idx.shape
    n_rows = b * t
    idx2 = idx.reshape(n_rows // SUB, SUB)
    out = _gather(idx2, embeddings, n_rows, 0)
    return out.reshape(b, t, DIM)


# speedup vs baseline: 5.0845x; 1.0530x over previous
import functools

import jax
import jax.numpy as jnp
from jax import lax
from jax.experimental import pallas as pl
from jax.experimental.pallas import tpu as pltpu
from jax.experimental.pallas import tpu_sc as plsc

DIM = 32
SUB = 128
NSUB = 8
CHUNK = SUB * NSUB
LANES = 16


@functools.partial(jax.jit, static_argnums=(2, 3))
def _gather(idx2, table, n_rows, padding_idx):
    info = plsc.get_sparse_core_info()
    nw = info.num_cores * info.num_subcores
    n_w = n_rows // nw
    nch = n_w // CHUNK
    assert nch % 2 == 0

    mesh = plsc.VectorSubcoreMesh(core_axis_name="c", subcore_axis_name="s")

    @functools.partial(
        pl.kernel,
        mesh=mesh,
        compiler_params=pltpu.CompilerParams(use_tc_tiling_on_sc=False),
        out_type=jax.ShapeDtypeStruct((n_rows, DIM), jnp.float32),
        scratch_types=[
            pltpu.VMEM((2, NSUB, SUB), jnp.int32),
            pltpu.VMEM((2, CHUNK, DIM), jnp.float32),
            pltpu.SemaphoreType.DMA,
            pltpu.SemaphoreType.DMA,
            pltpu.SemaphoreType.DMA,
            pltpu.SemaphoreType.DMA,
        ],
    )
    def k(idx_hbm, table_hbm, out_hbm, idx_v, rows_v,
          gsem0, gsem1, isem0, isem1):
        gsem = (gsem0, gsem1)
        isem = (isem0, isem1)
        c = lax.axis_index("c")
        s = lax.axis_index("s")
        wid = s * info.num_cores + c
        base = wid * n_w

        def idx_src(g):
            r0 = pl.multiple_of((base + g * CHUNK) // SUB, NSUB)
            return idx_hbm.at[pl.ds(r0, NSUB)]

        def fire_gathers(g, buf):
            return [
                pltpu.async_copy(
                    table_hbm.at[idx_v.at[buf, j]],
                    rows_v.at[buf, pl.ds(j * SUB, SUB)],
                    gsem[buf],
                )
                for j in range(NSUB)
            ]

        def detect_pad(buf):
            acc = idx_v[buf, 0, pl.ds(0, LANES)] == padding_idx
            for t in range(1, CHUNK // LANES):
                vec = idx_v[buf, t // (SUB // LANES),
                            pl.ds((t % (SUB // LANES)) * LANES, LANES)]
                acc = acc | (vec == padding_idx)
            acc_i = jnp.where(acc, 1, 0)
            pad = acc_i[0]
            for lane in range(1, LANES):
                pad = pad | acc_i[lane]
            return pad > 0

        def fix_pad(buf):
            zeros = jnp.zeros((LANES,), jnp.float32)

            def grp_fix(t, carry2):
                j = t // (SUB // LANES)
                off = (t % (SUB // LANES)) * LANES
                vec = idx_v[buf, j, pl.ds(off, LANES)]
                for lane in range(LANES):
                    row = t * LANES + lane

                    @pl.when(vec[lane] == padding_idx)
                    def _z(row=row):
                        rows_v[buf, row, pl.ds(0, LANES)] = zeros
                        rows_v[buf, row, pl.ds(LANES, LANES)] = zeros

                return carry2

            lax.fori_loop(0, CHUNK // LANES, grp_fix, 0)

        def process(g, buf):
            @pl.when(g + 1 < nch)
            def _next():
                pltpu.make_async_copy(
                    idx_src(g + 1), idx_v.at[1 - buf], isem[1 - buf]
                ).wait()
                fire_gathers(g + 1, 1 - buf)

            has_pad = detect_pad(buf)
            for j in range(NSUB):
                pltpu.make_async_copy(
                    table_hbm.at[idx_v.at[buf, j]],
                    rows_v.at[buf, pl.ds(j * SUB, SUB)],
                    gsem[buf],
                ).wait()

            @pl.when(has_pad)
            def _fix():
                fix_pad(buf)

            @pl.when(g + 2 < nch)
            def _pref():
                pltpu.async_copy(idx_src(g + 2), idx_v.at[buf], isem[buf])

            row0 = pl.multiple_of(base + g * CHUNK, CHUNK)
            pltpu.sync_copy(rows_v.at[buf], out_hbm.at[pl.ds(row0, CHUNK)])

        pltpu.sync_copy(idx_src(0), idx_v.at[0])
        fire_gathers(0, 0)
        pltpu.async_copy(idx_src(1), idx_v.at[1], isem[1])

        def pair(p, carry):
            process(2 * p, 0)
            process(2 * p + 1, 1)
            return carry

        lax.fori_loop(0, nch // 2, pair, 0)

    return k(idx2, table)


def kernel(idx, embeddings):
    b, t = idx.shape
    n_rows = b * t
    idx2 = idx.reshape(n_rows // SUB, SUB)
    out = _gather(idx2, embeddings, n_rows, 0)
    return out.reshape(b, t, DIM)
